# baseline (speedup 1.0000x reference)
import functools

import jax
import jax.numpy as jnp
from jax import lax
from jax.experimental import pallas as pl
from jax.experimental.pallas import tpu as pltpu
from jax.experimental.pallas import tpu_sc as plsc

N = 10000
NP = 10240
NC, NS = 2, 16
NW = NC * NS
CHUNK = 128
NBUF = 8
ROUNDS = 10
CPT = NBUF * ROUNDS
EPAD = NW * CHUNK * CPT
R_SLOW = 4
R_FAST = 16
MAXR = R_FAST
RPT = NP // NS
BLK = 1000
GRID = N // BLK

_mesh = lambda: plsc.VectorSubcoreMesh(core_axis_name="c", subcore_axis_name="s")


@jax.jit
def _sc_degree(dst_pad):
    @functools.partial(
        pl.kernel,
        out_type=jax.ShapeDtypeStruct((NC, NP), jnp.float32),
        mesh=_mesh(),
        compiler_params=pltpu.CompilerParams(use_tc_tiling_on_sc=False),
        scratch_types=[
            pltpu.VMEM((2, NBUF, CHUNK), jnp.int32),
            pltpu.VMEM((CHUNK,), jnp.float32),
            pltpu.VMEM((RPT,), jnp.float32),
            pltpu.VMEM_SHARED((NP,), jnp.float32),
            pltpu.SemaphoreType.DMA,
            pltpu.SemaphoreType.DMA,
        ],
    )
    def k(dst_hbm, out_hbm, idxb, ones_v, zero_v, acc_sh, sem_i, sem_s):
        cid = lax.axis_index("c")
        sid = lax.axis_index("s")
        wid = cid * NS + sid

        pltpu.async_copy(dst_hbm.at[wid, 0], idxb.at[0], sem_i)
        for i in range(CHUNK // 16):
            ones_v[pl.ds(i * 16, 16)] = jnp.full((16,), 1.0, jnp.float32)
        for i in range(RPT // 16):
            zero_v[pl.ds(i * 16, 16)] = jnp.zeros((16,), jnp.float32)
        pltpu.sync_copy(zero_v, acc_sh.at[pl.ds(sid * RPT, RPT)])
        plsc.subcore_barrier()

        def body(g2, _):
            for par in range(2):
                g = g2 * 2 + par
                pltpu.make_async_copy(dst_hbm.at[wid, 0], idxb.at[par],
                                      sem_i).wait()
                pltpu.async_copy(dst_hbm.at[wid, g + 1], idxb.at[1 - par],
                                 sem_i)
                sds = [pltpu.async_copy(ones_v, acc_sh.at[idxb.at[par, b]],
                                        sem_s, add=True)
                       for b in range(NBUF)]
                for d_ in sds:
                    d_.wait()
            return ()

        lax.fori_loop(0, ROUNDS // 2, body, ())
        pltpu.make_async_copy(dst_hbm.at[wid, 0], idxb.at[0], sem_i).wait()
        plsc.subcore_barrier()
        pltpu.sync_copy(acc_sh.at[pl.ds(sid * RPT, RPT)],
                        out_hbm.at[cid, pl.ds(sid * RPT, RPT)])

    return k(dst_pad)


def _make_sc_aggregate(D):
    @jax.jit
    def agg(rows, pairs):
        @functools.partial(
            pl.kernel,
            out_type=jax.ShapeDtypeStruct((NC, NP, D), jnp.float32),
            mesh=_mesh(),
            compiler_params=pltpu.CompilerParams(use_tc_tiling_on_sc=False),
            scratch_types=[
                pltpu.VMEM((2, NBUF, 2, CHUNK), jnp.int32),
                pltpu.VMEM((NBUF, CHUNK, D), jnp.float32),
                pltpu.VMEM((CHUNK, D), jnp.float32),
                pltpu.VMEM_SHARED((NP, D), jnp.float32),
                pltpu.SemaphoreType.DMA,
                pltpu.SemaphoreType.DMA,
                pltpu.SemaphoreType.DMA,
            ],
        )
        def k(rows_hbm, pairs_hbm, out_hbm, idxb, rows_v, zero_v, acc_sh,
              sem_i, sem_g, sem_s):
            cid = lax.axis_index("c")
            sid = lax.axis_index("s")
            wid = cid * NS + sid

            pltpu.async_copy(pairs_hbm.at[wid, 0], idxb.at[0], sem_i)

            def zfill(i, _):
                for c in range(D // 16):
                    zero_v[i, pl.ds(c * 16, 16)] = jnp.zeros((16,), jnp.float32)
                return ()

            lax.fori_loop(0, CHUNK, zfill, ())

            def zcopy(r, _):
                pltpu.sync_copy(zero_v, acc_sh.at[pl.ds(sid * RPT + r * CHUNK, CHUNK)])
                return ()

            lax.fori_loop(0, RPT // CHUNK, zcopy, ())
            plsc.subcore_barrier()

            def body(g2, _):
                for par in range(2):
                    g = g2 * 2 + par
                    pltpu.make_async_copy(pairs_hbm.at[wid, 0], idxb.at[par],
                                          sem_i).wait()
                    pltpu.async_copy(pairs_hbm.at[wid, g + 1], idxb.at[1 - par],
                                     sem_i)
                    gds = [pltpu.async_copy(rows_hbm.at[idxb.at[par, b, 0]],
                                            rows_v.at[b], sem_g)
                           for b in range(NBUF)]
                    for d_ in gds:
                        d_.wait()
                    sds = [pltpu.async_copy(rows_v.at[b],
                                            acc_sh.at[idxb.at[par, b, 1]],
                                            sem_s, add=True)
                           for b in range(NBUF)]
                    for d_ in sds:
                        d_.wait()
                return ()

            nrounds = jnp.where(cid == 0, R_FAST, R_SLOW)
            lax.fori_loop(0, nrounds // 2, body, ())
            pltpu.make_async_copy(pairs_hbm.at[wid, 0], idxb.at[0],
                                  sem_i).wait()
            plsc.subcore_barrier()
            pltpu.sync_copy(acc_sh.at[pl.ds(sid * RPT, RPT)],
                            out_hbm.at[cid, pl.ds(sid * RPT, RPT)])

        return k(rows, pairs)

    return agg


_sc_agg32 = _make_sc_aggregate(32)
_sc_agg64 = _make_sc_aggregate(64)


def _dinv(dp_blk):
    return lax.rsqrt(dp_blk[:, 0:1] + dp_blk[:, 1:2] + 1.0)


def _tc1_body(x_ref, w_ref, dp_ref, o_ref):
    dinv = _dinv(dp_ref[...])
    h = jnp.dot(x_ref[...], w_ref[...], preferred_element_type=jnp.float32)
    o_ref[...] = h * dinv


@jax.jit
def _tc1(x_p, W1, dp_t):
    return pl.pallas_call(
        _tc1_body,
        grid=(GRID,),
        in_specs=[
            pl.BlockSpec((BLK, 128), lambda i: (i, 0)),
            pl.BlockSpec((128, 32), lambda i: (0, 0)),
            pl.BlockSpec((BLK, 2), lambda i: (i, 0)),
        ],
        out_specs=pl.BlockSpec((BLK, 32), lambda i: (i, 0)),
        out_shape=jax.ShapeDtypeStruct((N, 32), jnp.float32),
    )(x_p, W1, dp_t)


def _tc2_body(a_ref, hs_ref, dp_ref, b1_ref, w2_ref, o_ref):
    dinv = _dinv(dp_ref[...])
    pre = dinv * (a_ref[0] + a_ref[1] + hs_ref[...]) + b1_ref[...]
    act = jnp.where(pre > 0, pre, jnp.exp(jnp.minimum(pre, 0.0)) - 1.0)
    g = jnp.dot(act, w2_ref[...], preferred_element_type=jnp.float32)
    o_ref[...] = g * dinv


@jax.jit
def _tc2(ap1, hs1, dp_t, b1r, W2):
    return pl.pallas_call(
        _tc2_body,
        grid=(GRID,),
        in_specs=[
            pl.BlockSpec((NC, BLK, 32), lambda i: (0, i, 0)),
            pl.BlockSpec((BLK, 32), lambda i: (i, 0)),
            pl.BlockSpec((BLK, 2), lambda i: (i, 0)),
            pl.BlockSpec((1, 32), lambda i: (0, 0)),
            pl.BlockSpec((32, 64), lambda i: (0, 0)),
        ],
        out_specs=pl.BlockSpec((BLK, 64), lambda i: (i, 0)),
        out_shape=jax.ShapeDtypeStruct((N, 64), jnp.float32),
    )(ap1, hs1, dp_t, b1r, W2)


def _tc3_body(a_ref, gs_ref, dp_ref, b2_ref, o_ref):
    dinv = _dinv(dp_ref[...])
    o_ref[...] = dinv * (a_ref[0] + a_ref[1] + gs_ref[...]) + b2_ref[...]


@jax.jit
def _tc3(ap2, gs, dp_t, b2r):
    return pl.pallas_call(
        _tc3_body,
        grid=(GRID,),
        in_specs=[
            pl.BlockSpec((NC, BLK, 64), lambda i: (0, i, 0)),
            pl.BlockSpec((BLK, 64), lambda i: (i, 0)),
            pl.BlockSpec((BLK, 2), lambda i: (i, 0)),
            pl.BlockSpec((1, 64), lambda i: (0, 0)),
        ],
        out_specs=pl.BlockSpec((BLK, 64), lambda i: (i, 0)),
        out_shape=jax.ShapeDtypeStruct((N, 64), jnp.float32),
    )(ap2, gs, dp_t, b2r)


def kernel(x, edge_index, W1, b1, W2, b2):
    e = edge_index.shape[1]
    src_pad = jnp.concatenate(
        [edge_index[0], jnp.zeros((EPAD - e,), jnp.int32)])
    dst_pad = jnp.concatenate(
        [edge_index[1], N + jnp.arange(EPAD - e, dtype=jnp.int32) % (NP - N)])

    def split(v):
        c = v.reshape(NW * CPT, CHUNK)
        nslow = NS * R_SLOW * NBUF
        s = c[:nslow].reshape(NS, R_SLOW, NBUF, CHUNK)
        f = c[nslow:].reshape(NS, R_FAST, NBUF, CHUNK)
        s = jnp.pad(s, ((0, 0), (0, MAXR + 1 - R_SLOW), (0, 0), (0, 0)))
        f = jnp.pad(f, ((0, 0), (0, 1), (0, 0), (0, 0)))
        return jnp.concatenate([f, s], axis=0)

    pairs = jnp.stack([split(src_pad), split(dst_pad)], axis=3)
    dchunks = jnp.pad(dst_pad.reshape(NW, ROUNDS, NBUF, CHUNK),
                      ((0, 0), (0, 1), (0, 0), (0, 0)))

    deg_parts = _sc_degree(dchunks)
    dp_t = deg_parts.T

    hs1 = _tc1(x, W1, dp_t)
    ap1 = _sc_agg32(hs1, pairs)
    gs = _tc2(ap1, hs1, dp_t, b1.reshape(1, 32), W2)
    ap2 = _sc_agg64(gs, pairs)
    return _tc3(ap2, gs, dp_t, b2.reshape(1, 64))


# speedup vs baseline: 1.3292x; 1.3292x over previous
import functools

import jax
import jax.numpy as jnp
from jax import lax
from jax.experimental import pallas as pl
from jax.experimental.pallas import tpu as pltpu
from jax.experimental.pallas import tpu_sc as plsc

N = 10000
NP = 10240
NC, NS = 2, 16
NW = NC * NS
CHUNK = 128
NBUF = 8
ROUNDS = 10
CPT = NBUF * ROUNDS
EPAD = NW * CHUNK * CPT
R_SLOW = 10
R_FAST = 10
MAXR = R_FAST
RPT = NP // NS
BLK = 1000
GRID = N // BLK

_mesh = lambda: plsc.VectorSubcoreMesh(core_axis_name="c", subcore_axis_name="s")


@jax.jit
def _sc_degree(dst_pad):
    @functools.partial(
        pl.kernel,
        out_type=jax.ShapeDtypeStruct((NC, NP), jnp.float32),
        mesh=_mesh(),
        compiler_params=pltpu.CompilerParams(use_tc_tiling_on_sc=False),
        scratch_types=[
            pltpu.VMEM((2, NBUF, CHUNK), jnp.int32),
            pltpu.VMEM((CHUNK,), jnp.float32),
            pltpu.VMEM((RPT,), jnp.float32),
            pltpu.VMEM_SHARED((NP,), jnp.float32),
            pltpu.SemaphoreType.DMA,
            pltpu.SemaphoreType.DMA,
        ],
    )
    def k(dst_hbm, out_hbm, idxb, ones_v, zero_v, acc_sh, sem_i, sem_s):
        cid = lax.axis_index("c")
        sid = lax.axis_index("s")
        wid = cid * NS + sid

        pltpu.async_copy(dst_hbm.at[wid, 0], idxb.at[0], sem_i)
        for i in range(CHUNK // 16):
            ones_v[pl.ds(i * 16, 16)] = jnp.full((16,), 1.0, jnp.float32)
        for i in range(RPT // 16):
            zero_v[pl.ds(i * 16, 16)] = jnp.zeros((16,), jnp.float32)
        pltpu.sync_copy(zero_v, acc_sh.at[pl.ds(sid * RPT, RPT)])
        plsc.subcore_barrier()

        def body(g2, _):
            for par in range(2):
                g = g2 * 2 + par
                pltpu.make_async_copy(dst_hbm.at[wid, 0], idxb.at[par],
                                      sem_i).wait()
                pltpu.async_copy(dst_hbm.at[wid, g + 1], idxb.at[1 - par],
                                 sem_i)
                sds = [pltpu.async_copy(ones_v, acc_sh.at[idxb.at[par, b]],
                                        sem_s, add=True)
                       for b in range(NBUF)]
                for d_ in sds:
                    d_.wait()
            return ()

        lax.fori_loop(0, ROUNDS // 2, body, ())
        pltpu.make_async_copy(dst_hbm.at[wid, 0], idxb.at[0], sem_i).wait()
        plsc.subcore_barrier()
        pltpu.sync_copy(acc_sh.at[pl.ds(sid * RPT, RPT)],
                        out_hbm.at[cid, pl.ds(sid * RPT, RPT)])

    return k(dst_pad)


def _make_sc_aggregate(D, stage=True):
    @jax.jit
    def agg(rows, pairs):
        @functools.partial(
            pl.kernel,
            out_type=pltpu.HBM((NC, NP, D), jnp.float32),
            mesh=_mesh(),
            compiler_params=pltpu.CompilerParams(use_tc_tiling_on_sc=False),
            scratch_types=[
                pltpu.VMEM((2, NBUF, 2, CHUNK), jnp.int32),
                pltpu.VMEM((NBUF, CHUNK, D), jnp.float32),
                pltpu.VMEM((CHUNK, D), jnp.float32),
                (pltpu.VMEM_SHARED((N, D), jnp.float32) if stage
                 else pltpu.VMEM_SHARED((8, D), jnp.float32)),
                pltpu.VMEM_SHARED((NP, D), jnp.float32),
                pltpu.SemaphoreType.DMA,
                pltpu.SemaphoreType.DMA,
                pltpu.SemaphoreType.DMA,
            ],
        )
        def k(rows_hbm, pairs_hbm, out_hbm, idxb, rows_v, zero_v, tab_sh,
              acc_sh, sem_i, sem_g, sem_s):
            cid = lax.axis_index("c")
            sid = lax.axis_index("s")
            wid = cid * NS + sid

            pltpu.async_copy(pairs_hbm.at[wid, 0], idxb.at[0], sem_i)
            if stage:
                pltpu.sync_copy(rows_hbm.at[pl.ds(sid * (N // NS), N // NS)],
                                tab_sh.at[pl.ds(sid * (N // NS), N // NS)])
            gather_src = tab_sh if stage else rows_hbm

            def zfill(i, _):
                for c in range(D // 16):
                    zero_v[i, pl.ds(c * 16, 16)] = jnp.zeros((16,), jnp.float32)
                return ()

            lax.fori_loop(0, CHUNK, zfill, ())

            def zcopy(r, _):
                pltpu.sync_copy(zero_v, acc_sh.at[pl.ds(sid * RPT + r * CHUNK, CHUNK)])
                return ()

            lax.fori_loop(0, RPT // CHUNK, zcopy, ())
            plsc.subcore_barrier()

            def body(g2, _):
                for par in range(2):
                    g = g2 * 2 + par
                    pltpu.make_async_copy(pairs_hbm.at[wid, 0], idxb.at[par],
                                          sem_i).wait()
                    pltpu.async_copy(pairs_hbm.at[wid, g + 1], idxb.at[1 - par],
                                     sem_i)
                    gds = [pltpu.async_copy(gather_src.at[idxb.at[par, b, 0]],
                                            rows_v.at[b], sem_g)
                           for b in range(NBUF)]
                    for d_ in gds:
                        d_.wait()
                    sds = [pltpu.async_copy(rows_v.at[b],
                                            acc_sh.at[idxb.at[par, b, 1]],
                                            sem_s, add=True)
                           for b in range(NBUF)]
                    for d_ in sds:
                        d_.wait()
                return ()

            nrounds = jnp.where(cid == 0, R_FAST, R_SLOW)
            lax.fori_loop(0, nrounds // 2, body, ())
            pltpu.make_async_copy(pairs_hbm.at[wid, 0], idxb.at[0],
                                  sem_i).wait()
            plsc.subcore_barrier()
            pltpu.sync_copy(acc_sh.at[pl.ds(sid * RPT, RPT)],
                            out_hbm.at[cid, pl.ds(sid * RPT, RPT)])

        return k(rows, pairs)

    return agg


_sc_agg32 = _make_sc_aggregate(32, stage=True)
_sc_agg64 = _make_sc_aggregate(64, stage=False)


def _dinv(dp_blk):
    return lax.rsqrt(dp_blk[:, 0:1] + dp_blk[:, 1:2] + 1.0)


def _tc1_body(x_ref, w_ref, dp_ref, o_ref):
    dinv = _dinv(dp_ref[...])
    h = jnp.dot(x_ref[...], w_ref[...], preferred_element_type=jnp.float32)
    o_ref[...] = h * dinv


@jax.jit
def _tc1(x_p, W1, dp_t):
    return pl.pallas_call(
        _tc1_body,
        grid=(GRID,),
        in_specs=[
            pl.BlockSpec((BLK, 128), lambda i: (i, 0)),
            pl.BlockSpec((128, 32), lambda i: (0, 0)),
            pl.BlockSpec((BLK, 2), lambda i: (i, 0)),
        ],
        out_specs=pl.BlockSpec((BLK, 32), lambda i: (i, 0)),
        out_shape=jax.ShapeDtypeStruct((N, 32), jnp.float32),
    )(x_p, W1, dp_t)


def _tc2_body(a_ref, hs_ref, dp_ref, b1_ref, w2_ref, o_ref):
    dinv = _dinv(dp_ref[...])
    pre = dinv * (a_ref[0] + a_ref[1] + hs_ref[...]) + b1_ref[...]
    act = jnp.where(pre > 0, pre, jnp.exp(jnp.minimum(pre, 0.0)) - 1.0)
    g = jnp.dot(act, w2_ref[...], preferred_element_type=jnp.float32)
    o_ref[...] = g * dinv


@jax.jit
def _tc2(ap1, hs1, dp_t, b1r, W2):
    return pl.pallas_call(
        _tc2_body,
        grid=(GRID,),
        in_specs=[
            pl.BlockSpec((NC, BLK, 32), lambda i: (0, i, 0)),
            pl.BlockSpec((BLK, 32), lambda i: (i, 0)),
            pl.BlockSpec((BLK, 2), lambda i: (i, 0)),
            pl.BlockSpec((1, 32), lambda i: (0, 0)),
            pl.BlockSpec((32, 64), lambda i: (0, 0)),
        ],
        out_specs=pl.BlockSpec((BLK, 64), lambda i: (i, 0)),
        out_shape=jax.ShapeDtypeStruct((N, 64), jnp.float32),
    )(ap1, hs1, dp_t, b1r, W2)


def _tc3_body(a_ref, gs_ref, dp_ref, b2_ref, o_ref):
    dinv = _dinv(dp_ref[...])
    o_ref[...] = dinv * (a_ref[0] + a_ref[1] + gs_ref[...]) + b2_ref[...]


@jax.jit
def _tc3(ap2, gs, dp_t, b2r):
    return pl.pallas_call(
        _tc3_body,
        grid=(GRID,),
        in_specs=[
            pl.BlockSpec((NC, BLK, 64), lambda i: (0, i, 0)),
            pl.BlockSpec((BLK, 64), lambda i: (i, 0)),
            pl.BlockSpec((BLK, 2), lambda i: (i, 0)),
            pl.BlockSpec((1, 64), lambda i: (0, 0)),
        ],
        out_specs=pl.BlockSpec((BLK, 64), lambda i: (i, 0)),
        out_shape=jax.ShapeDtypeStruct((N, 64), jnp.float32),
    )(ap2, gs, dp_t, b2r)


def kernel(x, edge_index, W1, b1, W2, b2):
    e = edge_index.shape[1]
    src_pad = jnp.concatenate(
        [edge_index[0], jnp.zeros((EPAD - e,), jnp.int32)])
    dst_pad = jnp.concatenate(
        [edge_index[1], N + jnp.arange(EPAD - e, dtype=jnp.int32) % (NP - N)])

    def split(v):
        c = v.reshape(NW * CPT, CHUNK)
        nslow = NS * R_SLOW * NBUF
        s = c[:nslow].reshape(NS, R_SLOW, NBUF, CHUNK)
        f = c[nslow:].reshape(NS, R_FAST, NBUF, CHUNK)
        s = jnp.pad(s, ((0, 0), (0, MAXR + 1 - R_SLOW), (0, 0), (0, 0)))
        f = jnp.pad(f, ((0, 0), (0, 1), (0, 0), (0, 0)))
        return jnp.concatenate([f, s], axis=0)

    pairs = jnp.stack([split(src_pad), split(dst_pad)], axis=3)
    dchunks = jnp.pad(dst_pad.reshape(NW, ROUNDS, NBUF, CHUNK),
                      ((0, 0), (0, 1), (0, 0), (0, 0)))

    deg_parts = _sc_degree(dchunks)
    dp_t = deg_parts.T

    hs1 = _tc1(x, W1, dp_t)
    ap1 = _sc_agg32(hs1, pairs)
    gs = _tc2(ap1, hs1, dp_t, b1.reshape(1, 32), W2)
    ap2 = _sc_agg64(gs, pairs)
    return _tc3(ap2, gs, dp_t, b2.reshape(1, 64))


# speedup vs baseline: 2.0650x; 1.5536x over previous
import functools

import jax
import jax.numpy as jnp
from jax import lax
from jax.experimental import pallas as pl
from jax.experimental.pallas import tpu as pltpu
from jax.experimental.pallas import tpu_sc as plsc

N = 10000
NP = 10240
NC, NS = 2, 16
NW = NC * NS
CHUNK = 128
NBUF = 8
ROUNDS = 10
CPT = NBUF * ROUNDS
EPAD = NW * CHUNK * CPT
R_SLOW = 10
R_FAST = 10
MAXR = R_FAST
RPT = NP // NS
BLK = 1000
GRID = N // BLK

_mesh = lambda: plsc.VectorSubcoreMesh(core_axis_name="c", subcore_axis_name="s")


@jax.jit
def _sc_degree(dst_pad):
    @functools.partial(
        pl.kernel,
        out_type=jax.ShapeDtypeStruct((NC, NP), jnp.float32),
        mesh=_mesh(),
        compiler_params=pltpu.CompilerParams(use_tc_tiling_on_sc=False),
        scratch_types=[
            pltpu.VMEM((2, NBUF, CHUNK), jnp.int32),
            pltpu.VMEM((CHUNK,), jnp.float32),
            pltpu.VMEM((RPT,), jnp.float32),
            pltpu.VMEM_SHARED((NP,), jnp.float32),
            pltpu.SemaphoreType.DMA,
            pltpu.SemaphoreType.DMA,
        ],
    )
    def k(dst_hbm, out_hbm, idxb, ones_v, zero_v, acc_sh, sem_i, sem_s):
        cid = lax.axis_index("c")
        sid = lax.axis_index("s")
        wid = cid * NS + sid

        pltpu.async_copy(dst_hbm.at[wid, 0], idxb.at[0], sem_i)
        for i in range(CHUNK // 16):
            ones_v[pl.ds(i * 16, 16)] = jnp.full((16,), 1.0, jnp.float32)
        for i in range(RPT // 16):
            zero_v[pl.ds(i * 16, 16)] = jnp.zeros((16,), jnp.float32)
        pltpu.sync_copy(zero_v, acc_sh.at[pl.ds(sid * RPT, RPT)])
        plsc.subcore_barrier()

        def body(g2, _):
            for par in range(2):
                g = g2 * 2 + par
                pltpu.make_async_copy(dst_hbm.at[wid, 0], idxb.at[par],
                                      sem_i).wait()
                pltpu.async_copy(dst_hbm.at[wid, g + 1], idxb.at[1 - par],
                                 sem_i)
                sds = [pltpu.async_copy(ones_v, acc_sh.at[idxb.at[par, b]],
                                        sem_s, add=True)
                       for b in range(NBUF)]
                for d_ in sds:
                    d_.wait()
            return ()

        lax.fori_loop(0, ROUNDS // 2, body, ())
        pltpu.make_async_copy(dst_hbm.at[wid, 0], idxb.at[0], sem_i).wait()
        plsc.subcore_barrier()
        pltpu.sync_copy(acc_sh.at[pl.ds(sid * RPT, RPT)],
                        out_hbm.at[cid, pl.ds(sid * RPT, RPT)])

    return k(dst_pad)


def _make_sc_aggregate(D, stage=True):
    @jax.jit
    def agg(rows, pairs):
        @functools.partial(
            pl.kernel,
            out_type=pltpu.HBM((NC, NP, D), jnp.float32),
            mesh=_mesh(),
            compiler_params=pltpu.CompilerParams(use_tc_tiling_on_sc=False),
            scratch_types=[
                pltpu.VMEM((2, NBUF, 2, CHUNK), jnp.int32),
                pltpu.VMEM((NBUF, CHUNK, D), jnp.float32),
                pltpu.VMEM((CHUNK, D), jnp.float32),
                (pltpu.VMEM_SHARED((N, D), jnp.float32) if stage
                 else pltpu.VMEM_SHARED((8, D), jnp.float32)),
                pltpu.VMEM_SHARED((NP, D), jnp.float32),
                pltpu.SemaphoreType.DMA,
                pltpu.SemaphoreType.DMA,
                pltpu.SemaphoreType.DMA,
            ],
        )
        def k(rows_hbm, pairs_hbm, out_hbm, idxb, rows_v, zero_v, tab_sh,
              acc_sh, sem_i, sem_g, sem_s):
            cid = lax.axis_index("c")
            sid = lax.axis_index("s")
            wid = cid * NS + sid

            pltpu.async_copy(pairs_hbm.at[wid, 0], idxb.at[0], sem_i)
            if stage:
                pltpu.sync_copy(rows_hbm.at[pl.ds(sid * (N // NS), N // NS)],
                                tab_sh.at[pl.ds(sid * (N // NS), N // NS)])
            gather_src = tab_sh if stage else rows_hbm

            def zfill(i, _):
                for c in range(D // 16):
                    zero_v[i, pl.ds(c * 16, 16)] = jnp.zeros((16,), jnp.float32)
                return ()

            lax.fori_loop(0, CHUNK, zfill, ())

            def zcopy(r, _):
                pltpu.sync_copy(zero_v, acc_sh.at[pl.ds(sid * RPT + r * CHUNK, CHUNK)])
                return ()

            lax.fori_loop(0, RPT // CHUNK, zcopy, ())
            plsc.subcore_barrier()

            def body(g2, _):
                for par in range(2):
                    g = g2 * 2 + par
                    pltpu.make_async_copy(pairs_hbm.at[wid, 0], idxb.at[par],
                                          sem_i).wait()
                    pltpu.async_copy(pairs_hbm.at[wid, g + 1], idxb.at[1 - par],
                                     sem_i)
                    gds = [pltpu.async_copy(gather_src.at[idxb.at[par, b, 0]],
                                            rows_v.at[b], sem_g)
                           for b in range(NBUF)]
                    for d_ in gds:
                        d_.wait()
                    sds = [pltpu.async_copy(rows_v.at[b],
                                            acc_sh.at[idxb.at[par, b, 1]],
                                            sem_s, add=True)
                           for b in range(NBUF)]
                    for d_ in sds:
                        d_.wait()
                return ()

            nrounds = jnp.where(cid == 0, R_FAST, R_SLOW)
            lax.fori_loop(0, nrounds // 2, body, ())
            pltpu.make_async_copy(pairs_hbm.at[wid, 0], idxb.at[0],
                                  sem_i).wait()
            plsc.subcore_barrier()
            pltpu.sync_copy(acc_sh.at[pl.ds(sid * RPT, RPT)],
                            out_hbm.at[cid, pl.ds(sid * RPT, RPT)])

        return k(rows, pairs)

    return agg


_sc_agg32 = _make_sc_aggregate(32, stage=True)


def _dinv(dp_blk):
    return lax.rsqrt(dp_blk[:, 0:1] + dp_blk[:, 1:2] + 1.0)


def _tc1_body(x_ref, w_ref, dp_ref, o_ref):
    dinv = _dinv(dp_ref[...])
    h = jnp.dot(x_ref[...], w_ref[...], preferred_element_type=jnp.float32)
    o_ref[...] = h * dinv


@jax.jit
def _tc1(x_p, W1, dp_t):
    return pl.pallas_call(
        _tc1_body,
        grid=(GRID,),
        in_specs=[
            pl.BlockSpec((BLK, 128), lambda i: (i, 0)),
            pl.BlockSpec((128, 32), lambda i: (0, 0)),
            pl.BlockSpec((BLK, 2), lambda i: (i, 0)),
        ],
        out_specs=pl.BlockSpec((BLK, 32), lambda i: (i, 0)),
        out_shape=jax.ShapeDtypeStruct((N, 32), jnp.float32),
    )(x_p, W1, dp_t)


def _tc2_body(a_ref, hs_ref, dp_ref, b1_ref, w2_ref, lo_ref, hi_ref):
    dinv = _dinv(dp_ref[...])
    pre = dinv * (a_ref[0] + a_ref[1] + hs_ref[...]) + b1_ref[...]
    act = jnp.where(pre > 0, pre, jnp.exp(jnp.minimum(pre, 0.0)) - 1.0)
    g = jnp.dot(act, w2_ref[...], preferred_element_type=jnp.float32)
    gs = g * dinv
    lo_ref[...] = gs[:, :32]
    hi_ref[...] = gs[:, 32:]


@jax.jit
def _tc2(ap1, hs1, dp_t, b1r, W2):
    return pl.pallas_call(
        _tc2_body,
        grid=(GRID,),
        in_specs=[
            pl.BlockSpec((NC, BLK, 32), lambda i: (0, i, 0)),
            pl.BlockSpec((BLK, 32), lambda i: (i, 0)),
            pl.BlockSpec((BLK, 2), lambda i: (i, 0)),
            pl.BlockSpec((1, 32), lambda i: (0, 0)),
            pl.BlockSpec((32, 64), lambda i: (0, 0)),
        ],
        out_specs=[pl.BlockSpec((BLK, 32), lambda i: (i, 0)),
                   pl.BlockSpec((BLK, 32), lambda i: (i, 0))],
        out_shape=[jax.ShapeDtypeStruct((N, 32), jnp.float32),
                   jax.ShapeDtypeStruct((N, 32), jnp.float32)],
    )(ap1, hs1, dp_t, b1r, W2)


def _tc3_body(alo_ref, ahi_ref, lo_ref, hi_ref, dp_ref, b2_ref, o_ref):
    dinv = _dinv(dp_ref[...])
    lo = alo_ref[0] + alo_ref[1] + lo_ref[...]
    hi = ahi_ref[0] + ahi_ref[1] + hi_ref[...]
    o_ref[...] = dinv * jnp.concatenate([lo, hi], axis=1) + b2_ref[...]


@jax.jit
def _tc3(ap_lo, ap_hi, gs_lo, gs_hi, dp_t, b2r):
    return pl.pallas_call(
        _tc3_body,
        grid=(GRID,),
        in_specs=[
            pl.BlockSpec((NC, BLK, 32), lambda i: (0, i, 0)),
            pl.BlockSpec((NC, BLK, 32), lambda i: (0, i, 0)),
            pl.BlockSpec((BLK, 32), lambda i: (i, 0)),
            pl.BlockSpec((BLK, 32), lambda i: (i, 0)),
            pl.BlockSpec((BLK, 2), lambda i: (i, 0)),
            pl.BlockSpec((1, 64), lambda i: (0, 0)),
        ],
        out_specs=pl.BlockSpec((BLK, 64), lambda i: (i, 0)),
        out_shape=jax.ShapeDtypeStruct((N, 64), jnp.float32),
    )(ap_lo, ap_hi, gs_lo, gs_hi, dp_t, b2r)


def kernel(x, edge_index, W1, b1, W2, b2):
    e = edge_index.shape[1]
    src_pad = jnp.concatenate(
        [edge_index[0], jnp.zeros((EPAD - e,), jnp.int32)])
    dst_pad = jnp.concatenate(
        [edge_index[1], N + jnp.arange(EPAD - e, dtype=jnp.int32) % (NP - N)])

    def split(v):
        c = v.reshape(NW * CPT, CHUNK)
        nslow = NS * R_SLOW * NBUF
        s = c[:nslow].reshape(NS, R_SLOW, NBUF, CHUNK)
        f = c[nslow:].reshape(NS, R_FAST, NBUF, CHUNK)
        s = jnp.pad(s, ((0, 0), (0, MAXR + 1 - R_SLOW), (0, 0), (0, 0)))
        f = jnp.pad(f, ((0, 0), (0, 1), (0, 0), (0, 0)))
        return jnp.concatenate([f, s], axis=0)

    pairs = jnp.stack([split(src_pad), split(dst_pad)], axis=3)
    dchunks = jnp.pad(dst_pad.reshape(NW, ROUNDS, NBUF, CHUNK),
                      ((0, 0), (0, 1), (0, 0), (0, 0)))

    deg_parts = _sc_degree(dchunks)
    dp_t = deg_parts.T

    hs1 = _tc1(x, W1, dp_t)
    ap1 = _sc_agg32(hs1, pairs)
    gs_lo, gs_hi = _tc2(ap1, hs1, dp_t, b1.reshape(1, 32), W2)
    ap_lo = _sc_agg32(gs_lo, pairs)
    ap_hi = _sc_agg32(gs_hi, pairs)
    return _tc3(ap_lo, ap_hi, gs_lo, gs_hi, dp_t, b2.reshape(1, 64))


# speedup vs baseline: 2.1904x; 1.0607x over previous
import functools

import jax
import jax.numpy as jnp
from jax import lax
from jax.experimental import pallas as pl
from jax.experimental.pallas import tpu as pltpu
from jax.experimental.pallas import tpu_sc as plsc

N = 10000
NP = 10240
NC, NS = 2, 16
NW = NC * NS
CHUNK = 128
NBUF = 8
ROUNDS = 10
CPT = NBUF * ROUNDS
EPAD = NW * CHUNK * CPT
R_SLOW = 10
R_FAST = 10
MAXR = R_FAST
RPT = NP // NS
BLK = 1000
GRID = N // BLK

_mesh = lambda: plsc.VectorSubcoreMesh(core_axis_name="c", subcore_axis_name="s")


@jax.jit
def _sc_degree(dst_pad):
    @functools.partial(
        pl.kernel,
        out_type=jax.ShapeDtypeStruct((NC, NP), jnp.float32),
        mesh=_mesh(),
        compiler_params=pltpu.CompilerParams(use_tc_tiling_on_sc=False),
        scratch_types=[
            pltpu.VMEM((2, NBUF, CHUNK), jnp.int32),
            pltpu.VMEM((CHUNK,), jnp.float32),
            pltpu.VMEM((RPT,), jnp.float32),
            pltpu.VMEM_SHARED((NP,), jnp.float32),
            pltpu.SemaphoreType.DMA,
            pltpu.SemaphoreType.DMA,
        ],
    )
    def k(dst_hbm, out_hbm, idxb, ones_v, zero_v, acc_sh, sem_i, sem_s):
        cid = lax.axis_index("c")
        sid = lax.axis_index("s")
        wid = cid * NS + sid

        pltpu.async_copy(dst_hbm.at[wid, 0], idxb.at[0], sem_i)
        for i in range(CHUNK // 16):
            ones_v[pl.ds(i * 16, 16)] = jnp.full((16,), 1.0, jnp.float32)
        for i in range(RPT // 16):
            zero_v[pl.ds(i * 16, 16)] = jnp.zeros((16,), jnp.float32)
        pltpu.sync_copy(zero_v, acc_sh.at[pl.ds(sid * RPT, RPT)])
        plsc.subcore_barrier()

        def body(g2, _):
            for par in range(2):
                g = g2 * 2 + par
                pltpu.make_async_copy(dst_hbm.at[wid, 0], idxb.at[par],
                                      sem_i).wait()
                pltpu.async_copy(dst_hbm.at[wid, g + 1], idxb.at[1 - par],
                                 sem_i)
                sds = [pltpu.async_copy(ones_v, acc_sh.at[idxb.at[par, b]],
                                        sem_s, add=True)
                       for b in range(NBUF)]
                for d_ in sds:
                    d_.wait()
            return ()

        lax.fori_loop(0, ROUNDS // 2, body, ())
        pltpu.make_async_copy(dst_hbm.at[wid, 0], idxb.at[0], sem_i).wait()
        plsc.subcore_barrier()
        pltpu.sync_copy(acc_sh.at[pl.ds(sid * RPT, RPT)],
                        out_hbm.at[cid, pl.ds(sid * RPT, RPT)])

    return k(dst_pad)


def _make_sc_aggregate(D, stage=True):
    @jax.jit
    def agg(rows, pairs):
        @functools.partial(
            pl.kernel,
            out_type=pltpu.HBM((NC, NP, D), jnp.float32),
            mesh=_mesh(),
            compiler_params=pltpu.CompilerParams(use_tc_tiling_on_sc=False),
            scratch_types=[
                pltpu.VMEM((2, NBUF, 2, CHUNK), jnp.int32),
                pltpu.VMEM((NBUF, CHUNK, D), jnp.float32),
                pltpu.VMEM((CHUNK, D), jnp.float32),
                (pltpu.VMEM_SHARED((N, D), jnp.float32) if stage
                 else pltpu.VMEM_SHARED((8, D), jnp.float32)),
                pltpu.VMEM_SHARED((NP, D), jnp.float32),
                pltpu.SemaphoreType.DMA,
                pltpu.SemaphoreType.DMA,
                pltpu.SemaphoreType.DMA,
            ],
        )
        def k(rows_hbm, pairs_hbm, out_hbm, idxb, rows_v, zero_v, tab_sh,
              acc_sh, sem_i, sem_g, sem_s):
            cid = lax.axis_index("c")
            sid = lax.axis_index("s")
            wid = cid * NS + sid

            pltpu.async_copy(pairs_hbm.at[wid, 0], idxb.at[0], sem_i)
            if stage:
                pltpu.sync_copy(rows_hbm.at[pl.ds(sid * (N // NS), N // NS)],
                                tab_sh.at[pl.ds(sid * (N // NS), N // NS)])
            gather_src = tab_sh if stage else rows_hbm

            def zfill(i, _):
                for c in range(D // 16):
                    zero_v[i, pl.ds(c * 16, 16)] = jnp.zeros((16,), jnp.float32)
                return ()

            lax.fori_loop(0, CHUNK, zfill, ())

            def zcopy(r, _):
                pltpu.sync_copy(zero_v, acc_sh.at[pl.ds(sid * RPT + r * CHUNK, CHUNK)])
                return ()

            lax.fori_loop(0, RPT // CHUNK, zcopy, ())
            plsc.subcore_barrier()

            def body(g2, _):
                for par in range(2):
                    g = g2 * 2 + par
                    pltpu.make_async_copy(pairs_hbm.at[wid, 0], idxb.at[par],
                                          sem_i).wait()
                    pltpu.async_copy(pairs_hbm.at[wid, g + 1], idxb.at[1 - par],
                                     sem_i)
                    gds = [pltpu.async_copy(gather_src.at[idxb.at[par, b, 0]],
                                            rows_v.at[b], sem_g)
                           for b in range(NBUF)]
                    sds = []
                    for b in range(NBUF):
                        gds[b].wait()
                        sds.append(pltpu.async_copy(
                            rows_v.at[b], acc_sh.at[idxb.at[par, b, 1]],
                            sem_s, add=True))
                    for d_ in sds:
                        d_.wait()
                return ()

            nrounds = jnp.where(cid == 0, R_FAST, R_SLOW)
            lax.fori_loop(0, nrounds // 2, body, ())
            pltpu.make_async_copy(pairs_hbm.at[wid, 0], idxb.at[0],
                                  sem_i).wait()
            plsc.subcore_barrier()
            pltpu.sync_copy(acc_sh.at[pl.ds(sid * RPT, RPT)],
                            out_hbm.at[cid, pl.ds(sid * RPT, RPT)])

        return k(rows, pairs)

    return agg


_sc_agg32 = _make_sc_aggregate(32, stage=True)


def _dinv(dp_blk):
    return lax.rsqrt(dp_blk[:, 0:1] + dp_blk[:, 1:2] + 1.0)


def _tc1_body(x_ref, w_ref, dp_ref, o_ref):
    dinv = _dinv(dp_ref[...])
    h = jnp.dot(x_ref[...], w_ref[...], preferred_element_type=jnp.float32)
    o_ref[...] = h * dinv


@jax.jit
def _tc1(x_p, W1, dp_t):
    return pl.pallas_call(
        _tc1_body,
        grid=(GRID,),
        in_specs=[
            pl.BlockSpec((BLK, 128), lambda i: (i, 0)),
            pl.BlockSpec((128, 32), lambda i: (0, 0)),
            pl.BlockSpec((BLK, 2), lambda i: (i, 0)),
        ],
        out_specs=pl.BlockSpec((BLK, 32), lambda i: (i, 0)),
        out_shape=jax.ShapeDtypeStruct((N, 32), jnp.float32),
    )(x_p, W1, dp_t)


def _tc2_body(a_ref, hs_ref, dp_ref, b1_ref, w2_ref, lo_ref, hi_ref):
    dinv = _dinv(dp_ref[...])
    pre = dinv * (a_ref[0] + a_ref[1] + hs_ref[...]) + b1_ref[...]
    act = jnp.where(pre > 0, pre, jnp.exp(jnp.minimum(pre, 0.0)) - 1.0)
    g = jnp.dot(act, w2_ref[...], preferred_element_type=jnp.float32)
    gs = g * dinv
    lo_ref[...] = gs[:, :32]
    hi_ref[...] = gs[:, 32:]


@jax.jit
def _tc2(ap1, hs1, dp_t, b1r, W2):
    return pl.pallas_call(
        _tc2_body,
        grid=(GRID,),
        in_specs=[
            pl.BlockSpec((NC, BLK, 32), lambda i: (0, i, 0)),
            pl.BlockSpec((BLK, 32), lambda i: (i, 0)),
            pl.BlockSpec((BLK, 2), lambda i: (i, 0)),
            pl.BlockSpec((1, 32), lambda i: (0, 0)),
            pl.BlockSpec((32, 64), lambda i: (0, 0)),
        ],
        out_specs=[pl.BlockSpec((BLK, 32), lambda i: (i, 0)),
                   pl.BlockSpec((BLK, 32), lambda i: (i, 0))],
        out_shape=[jax.ShapeDtypeStruct((N, 32), jnp.float32),
                   jax.ShapeDtypeStruct((N, 32), jnp.float32)],
    )(ap1, hs1, dp_t, b1r, W2)


def _tc3_body(alo_ref, ahi_ref, lo_ref, hi_ref, dp_ref, b2_ref, o_ref):
    dinv = _dinv(dp_ref[...])
    lo = alo_ref[0] + alo_ref[1] + lo_ref[...]
    hi = ahi_ref[0] + ahi_ref[1] + hi_ref[...]
    o_ref[...] = dinv * jnp.concatenate([lo, hi], axis=1) + b2_ref[...]


@jax.jit
def _tc3(ap_lo, ap_hi, gs_lo, gs_hi, dp_t, b2r):
    return pl.pallas_call(
        _tc3_body,
        grid=(GRID,),
        in_specs=[
            pl.BlockSpec((NC, BLK, 32), lambda i: (0, i, 0)),
            pl.BlockSpec((NC, BLK, 32), lambda i: (0, i, 0)),
            pl.BlockSpec((BLK, 32), lambda i: (i, 0)),
            pl.BlockSpec((BLK, 32), lambda i: (i, 0)),
            pl.BlockSpec((BLK, 2), lambda i: (i, 0)),
            pl.BlockSpec((1, 64), lambda i: (0, 0)),
        ],
        out_specs=pl.BlockSpec((BLK, 64), lambda i: (i, 0)),
        out_shape=jax.ShapeDtypeStruct((N, 64), jnp.float32),
    )(ap_lo, ap_hi, gs_lo, gs_hi, dp_t, b2r)


def kernel(x, edge_index, W1, b1, W2, b2):
    e = edge_index.shape[1]
    src_pad = jnp.concatenate(
        [edge_index[0], jnp.zeros((EPAD - e,), jnp.int32)])
    dst_pad = jnp.concatenate(
        [edge_index[1], N + jnp.arange(EPAD - e, dtype=jnp.int32) % (NP - N)])

    def split(v):
        c = v.reshape(NW * CPT, CHUNK)
        nslow = NS * R_SLOW * NBUF
        s = c[:nslow].reshape(NS, R_SLOW, NBUF, CHUNK)
        f = c[nslow:].reshape(NS, R_FAST, NBUF, CHUNK)
        s = jnp.pad(s, ((0, 0), (0, MAXR + 1 - R_SLOW), (0, 0), (0, 0)))
        f = jnp.pad(f, ((0, 0), (0, 1), (0, 0), (0, 0)))
        return jnp.concatenate([f, s], axis=0)

    pairs = jnp.stack([split(src_pad), split(dst_pad)], axis=3)
    dchunks = jnp.pad(dst_pad.reshape(NW, ROUNDS, NBUF, CHUNK),
                      ((0, 0), (0, 1), (0, 0), (0, 0)))

    deg_parts = _sc_degree(dchunks)
    dp_t = deg_parts.T

    hs1 = _tc1(x, W1, dp_t)
    ap1 = _sc_agg32(hs1, pairs)
    gs_lo, gs_hi = _tc2(ap1, hs1, dp_t, b1.reshape(1, 32), W2)
    ap_lo = _sc_agg32(gs_lo, pairs)
    ap_hi = _sc_agg32(gs_hi, pairs)
    return _tc3(ap_lo, ap_hi, gs_lo, gs_hi, dp_t, b2.reshape(1, 64))


# speedup vs baseline: 2.2078x; 1.0079x over previous
import functools

import jax
import jax.numpy as jnp
from jax import lax
from jax.experimental import pallas as pl
from jax.experimental.pallas import tpu as pltpu
from jax.experimental.pallas import tpu_sc as plsc

N = 10000
NP = 10240
NC, NS = 2, 16
NW = NC * NS
CHUNK = 128
NBUF = 8
ROUNDS = 10
CPT = NBUF * ROUNDS
EPAD = NW * CHUNK * CPT
R_SLOW = 10
R_FAST = 10
MAXR = R_FAST
RPT = NP // NS
BLK = 1000
GRID = N // BLK

_mesh = lambda: plsc.VectorSubcoreMesh(core_axis_name="c", subcore_axis_name="s")


@jax.jit
def _sc_degree(dst_pad):
    @functools.partial(
        pl.kernel,
        out_type=jax.ShapeDtypeStruct((NC, NP), jnp.float32),
        mesh=_mesh(),
        compiler_params=pltpu.CompilerParams(use_tc_tiling_on_sc=False),
        scratch_types=[
            pltpu.VMEM((2, NBUF, CHUNK), jnp.int32),
            pltpu.VMEM((CHUNK,), jnp.float32),
            pltpu.VMEM((RPT,), jnp.float32),
            pltpu.VMEM_SHARED((NP,), jnp.float32),
            pltpu.SemaphoreType.DMA,
            pltpu.SemaphoreType.DMA,
        ],
    )
    def k(dst_hbm, out_hbm, idxb, ones_v, zero_v, acc_sh, sem_i, sem_s):
        cid = lax.axis_index("c")
        sid = lax.axis_index("s")
        wid = cid * NS + sid

        pltpu.async_copy(dst_hbm.at[wid, 0], idxb.at[0], sem_i)
        for i in range(CHUNK // 16):
            ones_v[pl.ds(i * 16, 16)] = jnp.full((16,), 1.0, jnp.float32)
        for i in range(RPT // 16):
            zero_v[pl.ds(i * 16, 16)] = jnp.zeros((16,), jnp.float32)
        pltpu.sync_copy(zero_v, acc_sh.at[pl.ds(sid * RPT, RPT)])
        plsc.subcore_barrier()

        def body(g2, _):
            for par in range(2):
                g = g2 * 2 + par
                pltpu.make_async_copy(dst_hbm.at[wid, 0], idxb.at[par],
                                      sem_i).wait()
                pltpu.async_copy(dst_hbm.at[wid, g + 1], idxb.at[1 - par],
                                 sem_i)
                sds = [pltpu.async_copy(ones_v, acc_sh.at[idxb.at[par, b]],
                                        sem_s, add=True)
                       for b in range(NBUF)]
                for d_ in sds:
                    d_.wait()
            return ()

        lax.fori_loop(0, ROUNDS // 2, body, ())
        pltpu.make_async_copy(dst_hbm.at[wid, 0], idxb.at[0], sem_i).wait()
        plsc.subcore_barrier()
        pltpu.sync_copy(acc_sh.at[pl.ds(sid * RPT, RPT)],
                        out_hbm.at[cid, pl.ds(sid * RPT, RPT)])

    return k(dst_pad)


def _make_sc_aggregate(D, stage=True):
    @jax.jit
    def agg(rows, pairs):
        @functools.partial(
            pl.kernel,
            out_type=pltpu.HBM((NC, NP, D), jnp.float32),
            mesh=_mesh(),
            compiler_params=pltpu.CompilerParams(use_tc_tiling_on_sc=False),
            scratch_types=[
                pltpu.VMEM((2, NBUF, 2, CHUNK), jnp.int32),
                pltpu.VMEM((NBUF, CHUNK, D), jnp.float32),
                pltpu.VMEM((CHUNK, D), jnp.float32),
                (pltpu.VMEM_SHARED((N, D), jnp.float32) if stage
                 else pltpu.VMEM_SHARED((8, D), jnp.float32)),
                pltpu.VMEM_SHARED((NP, D), jnp.float32),
                pltpu.SemaphoreType.DMA,
                pltpu.SemaphoreType.DMA,
                pltpu.SemaphoreType.DMA,
            ],
        )
        def k(rows_hbm, pairs_hbm, out_hbm, idxb, rows_v, zero_v, tab_sh,
              acc_sh, sem_i, sem_g, sem_s):
            cid = lax.axis_index("c")
            sid = lax.axis_index("s")
            wid = cid * NS + sid

            pltpu.async_copy(pairs_hbm.at[wid, 0], idxb.at[0], sem_i)
            if stage:
                pltpu.sync_copy(rows_hbm.at[pl.ds(sid * (N // NS), N // NS)],
                                tab_sh.at[pl.ds(sid * (N // NS), N // NS)])
            gather_src = tab_sh if stage else rows_hbm

            def zfill(i, _):
                for c in range(D // 16):
                    zero_v[i, pl.ds(c * 16, 16)] = jnp.zeros((16,), jnp.float32)
                return ()

            lax.fori_loop(0, CHUNK, zfill, ())

            def zcopy(r, _):
                pltpu.sync_copy(zero_v, acc_sh.at[pl.ds(sid * RPT + r * CHUNK, CHUNK)])
                return ()

            lax.fori_loop(0, RPT // CHUNK, zcopy, ())
            plsc.subcore_barrier()

            def body(g2, _):
                for par in range(2):
                    g = g2 * 2 + par
                    pltpu.make_async_copy(pairs_hbm.at[wid, 0], idxb.at[par],
                                          sem_i).wait()
                    pltpu.async_copy(pairs_hbm.at[wid, g + 1], idxb.at[1 - par],
                                     sem_i)
                    gds = [pltpu.async_copy(gather_src.at[idxb.at[par, b, 0]],
                                            rows_v.at[b], sem_g)
                           for b in range(NBUF)]
                    sds = []
                    for b in range(NBUF):
                        gds[b].wait()
                        sds.append(pltpu.async_copy(
                            rows_v.at[b], acc_sh.at[idxb.at[par, b, 1]],
                            sem_s, add=True))
                    for d_ in sds:
                        d_.wait()
                return ()

            nrounds = jnp.where(cid == 0, R_FAST, R_SLOW)
            lax.fori_loop(0, nrounds // 2, body, ())
            pltpu.make_async_copy(pairs_hbm.at[wid, 0], idxb.at[0],
                                  sem_i).wait()
            plsc.subcore_barrier()
            pltpu.sync_copy(acc_sh.at[pl.ds(sid * RPT, RPT)],
                            out_hbm.at[cid, pl.ds(sid * RPT, RPT)])

        return k(rows, pairs)

    return agg


_sc_agg32 = _make_sc_aggregate(32, stage=True)


def _dinv(dp_blk):
    return lax.rsqrt(dp_blk[:, 0:1] + dp_blk[:, 1:2] + 1.0)


def _tc1_body(x_ref, w_ref, dp_ref, o_ref):
    dinv = _dinv(dp_ref[...][:N])
    h = jnp.dot(x_ref[...], w_ref[...], preferred_element_type=jnp.float32)
    o_ref[...] = h * dinv


@jax.jit
def _tc1(x_p, W1, dp_t):
    return pl.pallas_call(
        _tc1_body,
        out_shape=jax.ShapeDtypeStruct((N, 32), jnp.float32),
    )(x_p, W1, dp_t)


def _tc2_body(a_ref, hs_ref, dp_ref, b1_ref, w2_ref, lo_ref, hi_ref):
    dinv = _dinv(dp_ref[...][:N])
    pre = dinv * (a_ref[0, :N] + a_ref[1, :N] + hs_ref[...]) + b1_ref[...]
    act = jnp.where(pre > 0, pre, jnp.exp(jnp.minimum(pre, 0.0)) - 1.0)
    g = jnp.dot(act, w2_ref[...], preferred_element_type=jnp.float32)
    gs = g * dinv
    lo_ref[...] = gs[:, :32]
    hi_ref[...] = gs[:, 32:]


@jax.jit
def _tc2(ap1, hs1, dp_t, b1r, W2):
    return pl.pallas_call(
        _tc2_body,
        out_shape=[jax.ShapeDtypeStruct((N, 32), jnp.float32),
                   jax.ShapeDtypeStruct((N, 32), jnp.float32)],
    )(ap1, hs1, dp_t, b1r, W2)


def _tc3_body(alo_ref, ahi_ref, lo_ref, hi_ref, dp_ref, b2_ref, o_ref):
    dinv = _dinv(dp_ref[...][:N])
    lo = alo_ref[0, :N] + alo_ref[1, :N] + lo_ref[...]
    hi = ahi_ref[0, :N] + ahi_ref[1, :N] + hi_ref[...]
    o_ref[...] = dinv * jnp.concatenate([lo, hi], axis=1) + b2_ref[...]


@jax.jit
def _tc3(ap_lo, ap_hi, gs_lo, gs_hi, dp_t, b2r):
    return pl.pallas_call(
        _tc3_body,
        out_shape=jax.ShapeDtypeStruct((N, 64), jnp.float32),
    )(ap_lo, ap_hi, gs_lo, gs_hi, dp_t, b2r)


def kernel(x, edge_index, W1, b1, W2, b2):
    e = edge_index.shape[1]
    src_pad = jnp.concatenate(
        [edge_index[0], jnp.zeros((EPAD - e,), jnp.int32)])
    dst_pad = jnp.concatenate(
        [edge_index[1], N + jnp.arange(EPAD - e, dtype=jnp.int32) % (NP - N)])

    def split(v):
        c = v.reshape(NW * CPT, CHUNK)
        nslow = NS * R_SLOW * NBUF
        s = c[:nslow].reshape(NS, R_SLOW, NBUF, CHUNK)
        f = c[nslow:].reshape(NS, R_FAST, NBUF, CHUNK)
        s = jnp.pad(s, ((0, 0), (0, MAXR + 1 - R_SLOW), (0, 0), (0, 0)))
        f = jnp.pad(f, ((0, 0), (0, 1), (0, 0), (0, 0)))
        return jnp.concatenate([f, s], axis=0)

    pairs = jnp.stack([split(src_pad), split(dst_pad)], axis=3)
    dchunks = jnp.pad(dst_pad.reshape(NW, ROUNDS, NBUF, CHUNK),
                      ((0, 0), (0, 1), (0, 0), (0, 0)))

    deg_parts = _sc_degree(dchunks)
    dp_t = deg_parts.T

    hs1 = _tc1(x, W1, dp_t)
    ap1 = _sc_agg32(hs1, pairs)
    gs_lo, gs_hi = _tc2(ap1, hs1, dp_t, b1.reshape(1, 32), W2)
    ap_lo = _sc_agg32(gs_lo, pairs)
    ap_hi = _sc_agg32(gs_hi, pairs)
    return _tc3(ap_lo, ap_hi, gs_lo, gs_hi, dp_t, b2.reshape(1, 64))
